# Initial kernel scaffold; baseline (speedup 1.0000x reference)
import jax
import jax.numpy as jnp
from jax.experimental import pallas as pl


def kernel(h, edge_index0, edge_index1, W0, al0, ar0, W1, al1, ar1):
    raise NotImplementedError("write your pallas kernel here")


# speedup vs baseline: 15.1663x; 15.1663x over previous
import functools

import jax
import jax.numpy as jnp
from jax import lax
from jax.experimental import pallas as pl
from jax.experimental.pallas import tpu as pltpu
from jax.experimental.pallas import tpu_sc as plsc

N = 10000
E = 160000
DIN = 256
DOUT = 256
H = 128
NEG_SLOPE = 0.2

NC = 2
NS = 16
CHK = 128
NR = 80
EPS = NR * CHK
EP = NS * EPS
PADN = EP - E
NP = 10112
RPT = NP // NS

_MESH = plsc.VectorSubcoreMesh(
    core_axis_name="c", subcore_axis_name="s", num_cores=NC, num_subcores=NS)


def _tc_body(h_ref, w_ref, a_ref, tbl_ref, sc_ref):
    hb = h_ref[...]
    f0 = jnp.dot(hb, w_ref[0], preferred_element_type=jnp.float32)
    f1 = jnp.dot(hb, w_ref[1], preferred_element_type=jnp.float32)
    tbl_ref[0] = f0[:, :H]
    tbl_ref[1] = f0[:, H:]
    tbl_ref[2] = f1[:, :H]
    tbl_ref[3] = f1[:, H:]
    el0 = jnp.dot(f0, a_ref[0, 0], preferred_element_type=jnp.float32)
    er0 = jnp.dot(f0, a_ref[0, 1], preferred_element_type=jnp.float32)
    el1 = jnp.dot(f1, a_ref[1, 0], preferred_element_type=jnp.float32)
    er1 = jnp.dot(f1, a_ref[1, 1], preferred_element_type=jnp.float32)
    sc_ref[...] = jnp.stack([el0, er0, el1, er1], axis=1)


_BN = 1000


def _tc_feat(h, Ws, As):
    return pl.pallas_call(
        _tc_body,
        grid=(N // _BN,),
        in_specs=[
            pl.BlockSpec((_BN, DIN), lambda i: (i, 0)),
            pl.BlockSpec((2, DIN, DOUT), lambda i: (0, 0, 0)),
            pl.BlockSpec((2, 2, DOUT), lambda i: (0, 0, 0)),
        ],
        out_specs=[
            pl.BlockSpec((4, _BN, H), lambda i: (0, i, 0)),
            pl.BlockSpec((_BN, 4), lambda i: (i, 0)),
        ],
        out_shape=[
            jax.ShapeDtypeStruct((4, N, H), jnp.float32),
            jax.ShapeDtypeStruct((N, 4), jnp.float32),
        ],
    )(h, Ws, As)


def _ka_body(el0, er0, el1, er1, srcp, dstp, alpha_out,
             denom_sp, el_v, er_v, src_v, dst_v, ee_v, denom_v):
    c = lax.axis_index("c")
    s = lax.axis_index("s")

    @pl.when(s == 0)
    def _zero():
        def zb(i, _):
            denom_v[pl.ds(i * 16, 16)] = jnp.zeros((16,), jnp.float32)
            return 0
        lax.fori_loop(0, N // 16, zb, 0)
        pltpu.sync_copy(denom_v, denom_sp)

    @pl.when(c == 0)
    def _load0():
        pltpu.sync_copy(el0, el_v)
        pltpu.sync_copy(er0, er_v)

    @pl.when(c == 1)
    def _load1():
        pltpu.sync_copy(el1, el_v)
        pltpu.sync_copy(er1, er_v)
    pltpu.sync_copy(srcp.at[c, s], src_v)
    pltpu.sync_copy(dstp.at[c, s], dst_v)
    plsc.subcore_barrier()

    base = s * EPS

    def row(j, _):
        for k in range(CHK // 16):
            srcv = src_v[j, pl.ds(k * 16, 16)]
            dstv = dst_v[j, pl.ds(k * 16, 16)]
            e = plsc.load_gather(el_v, [srcv]) + plsc.load_gather(er_v, [dstv])
            e = jnp.where(e >= 0.0, e, e * NEG_SLOPE)
            ee = jnp.exp(e)
            eid = base + j * CHK + k * 16 + lax.iota(jnp.int32, 16)
            ee = jnp.where(eid < E, ee, 0.0)
            ee_v[j, pl.ds(k * 16, 16)] = ee
        pltpu.sync_copy(ee_v.at[j], denom_sp.at[dst_v.at[j]], add=True)
        return 0

    lax.fori_loop(0, NR, row, 0)
    plsc.subcore_barrier()
    pltpu.sync_copy(denom_sp, denom_v)

    def row2(j, _):
        for k in range(CHK // 16):
            dstv = dst_v[j, pl.ds(k * 16, 16)]
            dg = plsc.load_gather(denom_v, [dstv])
            a = ee_v[j, pl.ds(k * 16, 16)] / (dg + 1e-9) * 0.5
            ee_v[j, pl.ds(k * 16, 16)] = a
        return 0

    lax.fori_loop(0, NR, row2, 0)
    pltpu.sync_copy(ee_v, alpha_out.at[c, s])


_SC_PARAMS = pltpu.CompilerParams(needs_layout_passes=False)

_ka = functools.partial(
    pl.kernel,
    out_type=jax.ShapeDtypeStruct((NC, NS, NR, CHK), jnp.float32),
    mesh=_MESH,
    compiler_params=_SC_PARAMS,
    scratch_types=[
        pltpu.VMEM_SHARED((N,), jnp.float32),
        pltpu.VMEM((N,), jnp.float32),
        pltpu.VMEM((N,), jnp.float32),
        pltpu.VMEM((NR, CHK), jnp.int32),
        pltpu.VMEM((NR, CHK), jnp.int32),
        pltpu.VMEM((NR, CHK), jnp.float32),
        pltpu.VMEM((N,), jnp.float32),
    ],
)(_ka_body)


def _kb_body(tbl4, srcp, dstp, alphap, outh,
             accum_sp, src_v, dst_v, alpha_v, rows_v):
    c = lax.axis_index("c")
    s = lax.axis_index("s")

    def zb(i, _):
        for u in range(H // 16):
            rows_v[i, pl.ds(u * 16, 16)] = jnp.zeros((16,), jnp.float32)
        return 0
    lax.fori_loop(0, CHK, zb, 0)
    row0 = s * RPT
    for i in range(5):
        sz = CHK if i < 4 else RPT - 4 * CHK
        pltpu.sync_copy(rows_v.at[pl.ds(0, sz)],
                        accum_sp.at[pl.ds(row0 + i * CHK, sz)])
    plsc.subcore_barrier()

    for m in range(2):
        pltpu.sync_copy(srcp.at[m, s], src_v)
        pltpu.sync_copy(dstp.at[m, s], dst_v)
        pltpu.sync_copy(alphap.at[m, s], alpha_v)
        off = (2 * m) * N + c * N

        def row(j, _):
            for u in range(CHK // 16):
                src_v[j, pl.ds(u * 16, 16)] = src_v[j, pl.ds(u * 16, 16)] + off
            pltpu.sync_copy(tbl4.at[src_v.at[j]], rows_v)

            def scale(k, _):
                ab = plsc.load_gather(
                    alpha_v, [jnp.full((16,), j, jnp.int32),
                              jnp.full((16,), k, jnp.int32)])
                for u in range(H // 16):
                    rows_v[k, pl.ds(u * 16, 16)] = (
                        rows_v[k, pl.ds(u * 16, 16)] * ab)
                return 0

            lax.fori_loop(0, CHK, scale, 0, unroll=4)
            pltpu.sync_copy(rows_v, accum_sp.at[dst_v.at[j]], add=True)
            return 0

        lax.fori_loop(0, NR, row, 0)

    plsc.subcore_barrier()
    for i in range(5):
        sz = CHK if i < 4 else RPT - 4 * CHK
        pltpu.sync_copy(accum_sp.at[pl.ds(row0 + i * CHK, sz)],
                        outh.at[c, pl.ds(row0 + i * CHK, sz)])


_kb = functools.partial(
    pl.kernel,
    out_type=jax.ShapeDtypeStruct((NC, NP, H), jnp.float32),
    mesh=_MESH,
    compiler_params=_SC_PARAMS,
    scratch_types=[
        pltpu.VMEM_SHARED((NP, H), jnp.float32),
        pltpu.VMEM((NR, CHK), jnp.int32),
        pltpu.VMEM((NR, CHK), jnp.int32),
        pltpu.VMEM((NR, CHK), jnp.float32),
        pltpu.VMEM((CHK, H), jnp.float32),
    ],
)(_kb_body)


def kernel(h, edge_index0, edge_index1, W0, al0, ar0, W1, al1, ar1):
    Ws = jnp.stack([W0, W1])
    As = jnp.stack([jnp.stack([al0, ar0]), jnp.stack([al1, ar1])])
    tbl, sc = _tc_feat(h, Ws, As)

    pad = (jnp.arange(PADN, dtype=jnp.int32) * 97) % N

    def prep(ei):
        sfull = jnp.concatenate([ei[0], pad]).reshape(NS, NR, CHK)
        dfull = jnp.concatenate([ei[1], pad]).reshape(NS, NR, CHK)
        return sfull, dfull

    s0, d0 = prep(edge_index0)
    s1, d1 = prep(edge_index1)
    srcp = jnp.stack([s0, s1])
    dstp = jnp.stack([d0, d1])

    alpha = _ka(sc[:, 0], sc[:, 1], sc[:, 2], sc[:, 3], srcp, dstp)
    outh = _kb(tbl.reshape(4 * N, H), srcp, dstp, alpha)
    return jnp.concatenate([outh[0, :N], outh[1, :N]], axis=1)


# speedup vs baseline: 21.7829x; 1.4363x over previous
import functools

import jax
import jax.numpy as jnp
from jax import lax
from jax.experimental import pallas as pl
from jax.experimental.pallas import tpu as pltpu
from jax.experimental.pallas import tpu_sc as plsc

N = 10000
E = 160000
DIN = 256
DOUT = 256
H = 128
NEG_SLOPE = 0.2

NC = 2
NS = 16
CHK = 128
NR = 80
EPS = NR * CHK
EP = NS * EPS
PADN = EP - E
NP = 10112
RPT = NP // NS

_MESH = plsc.VectorSubcoreMesh(
    core_axis_name="c", subcore_axis_name="s", num_cores=NC, num_subcores=NS)


def _tc_body(h_ref, w_ref, a_ref, tbl_ref, sc_ref):
    hb = h_ref[...]
    f0 = jnp.dot(hb, w_ref[0], preferred_element_type=jnp.float32)
    f1 = jnp.dot(hb, w_ref[1], preferred_element_type=jnp.float32)
    tbl_ref[0] = f0[:, :H]
    tbl_ref[1] = f0[:, H:]
    tbl_ref[2] = f1[:, :H]
    tbl_ref[3] = f1[:, H:]
    el0 = jnp.dot(f0, a_ref[0, 0], preferred_element_type=jnp.float32)
    er0 = jnp.dot(f0, a_ref[0, 1], preferred_element_type=jnp.float32)
    el1 = jnp.dot(f1, a_ref[1, 0], preferred_element_type=jnp.float32)
    er1 = jnp.dot(f1, a_ref[1, 1], preferred_element_type=jnp.float32)
    sc_ref[...] = jnp.stack([el0, er0, el1, er1], axis=1)


_BN = 1000


def _tc_feat(h, Ws, As):
    return pl.pallas_call(
        _tc_body,
        grid=(N // _BN,),
        in_specs=[
            pl.BlockSpec((_BN, DIN), lambda i: (i, 0)),
            pl.BlockSpec((2, DIN, DOUT), lambda i: (0, 0, 0)),
            pl.BlockSpec((2, 2, DOUT), lambda i: (0, 0, 0)),
        ],
        out_specs=[
            pl.BlockSpec((4, _BN, H), lambda i: (0, i, 0)),
            pl.BlockSpec((_BN, 4), lambda i: (i, 0)),
        ],
        out_shape=[
            jax.ShapeDtypeStruct((4, N, H), jnp.float32),
            jax.ShapeDtypeStruct((N, 4), jnp.float32),
        ],
    )(h, Ws, As)


def _ka_body(el0, er0, el1, er1, srcp, dstp, alpha_out,
             denom_sp, el_v, er_v, src_v, dst_v, ee_v, denom_v):
    c = lax.axis_index("c")
    s = lax.axis_index("s")

    @pl.when(s == 0)
    def _zero():
        def zb(i, _):
            denom_v[pl.ds(i * 16, 16)] = jnp.zeros((16,), jnp.float32)
            return 0
        lax.fori_loop(0, N // 16, zb, 0)
        pltpu.sync_copy(denom_v, denom_sp)

    @pl.when(c == 0)
    def _load0():
        pltpu.sync_copy(el0, el_v)
        pltpu.sync_copy(er0, er_v)

    @pl.when(c == 1)
    def _load1():
        pltpu.sync_copy(el1, el_v)
        pltpu.sync_copy(er1, er_v)
    pltpu.sync_copy(srcp.at[c, s], src_v)
    pltpu.sync_copy(dstp.at[c, s], dst_v)
    plsc.subcore_barrier()

    base = s * EPS

    def row(j, _):
        for k in range(CHK // 16):
            srcv = src_v[j, pl.ds(k * 16, 16)]
            dstv = dst_v[j, pl.ds(k * 16, 16)]
            e = plsc.load_gather(el_v, [srcv]) + plsc.load_gather(er_v, [dstv])
            e = jnp.where(e >= 0.0, e, e * NEG_SLOPE)
            ee = jnp.exp(e)
            eid = base + j * CHK + k * 16 + lax.iota(jnp.int32, 16)
            ee = jnp.where(eid < E, ee, 0.0)
            ee_v[j, pl.ds(k * 16, 16)] = ee
        pltpu.sync_copy(ee_v.at[j], denom_sp.at[dst_v.at[j]], add=True)
        return 0

    lax.fori_loop(0, NR, row, 0)
    plsc.subcore_barrier()
    pltpu.sync_copy(denom_sp, denom_v)

    def row2(j, _):
        for k in range(CHK // 16):
            dstv = dst_v[j, pl.ds(k * 16, 16)]
            dg = plsc.load_gather(denom_v, [dstv])
            a = ee_v[j, pl.ds(k * 16, 16)] / (dg + 1e-9) * 0.5
            ee_v[j, pl.ds(k * 16, 16)] = a
        return 0

    lax.fori_loop(0, NR, row2, 0)
    pltpu.sync_copy(ee_v, alpha_out.at[c, s])


_SC_PARAMS = pltpu.CompilerParams(needs_layout_passes=False)

_ka = functools.partial(
    pl.kernel,
    out_type=jax.ShapeDtypeStruct((NC, NS, NR, CHK), jnp.float32),
    mesh=_MESH,
    compiler_params=_SC_PARAMS,
    scratch_types=[
        pltpu.VMEM_SHARED((N,), jnp.float32),
        pltpu.VMEM((N,), jnp.float32),
        pltpu.VMEM((N,), jnp.float32),
        pltpu.VMEM((NR, CHK), jnp.int32),
        pltpu.VMEM((NR, CHK), jnp.int32),
        pltpu.VMEM((NR, CHK), jnp.float32),
        pltpu.VMEM((N,), jnp.float32),
    ],
)(_ka_body)


_NB = 2
_NH = NR // 2


def _kb_body(tbl4, srcp, dstp, alphap, outh,
             accum_sp, src_v, dst_v, alpha_v,
             b0, b1, gs0, gs1, ss0, ss1):
    c = lax.axis_index("c")
    s = lax.axis_index("s")
    bufs = [b0, b1]
    gsems = [gs0, gs1]
    ssems = [ss0, ss1]

    def zb(i, _):
        for u in range(H // 16):
            b0[i, pl.ds(u * 16, 16)] = jnp.zeros((16,), jnp.float32)
        return 0
    lax.fori_loop(0, CHK, zb, 0)
    row0 = s * RPT
    for i in range(5):
        sz = CHK if i < 4 else RPT - 4 * CHK
        pltpu.sync_copy(b0.at[pl.ds(0, sz)],
                        accum_sp.at[pl.ds(row0 + i * CHK, sz)])
    plsc.subcore_barrier()

    for m in range(2):
        off = (2 * m) * N + c * N
        for hh in range(2):
            pltpu.sync_copy(srcp.at[m, s, pl.ds(hh * _NH, _NH)], src_v)
            pltpu.sync_copy(dstp.at[m, s, pl.ds(hh * _NH, _NH)], dst_v)
            pltpu.sync_copy(alphap.at[m, s, pl.ds(hh * _NH, _NH)], alpha_v)

            def issue_gather(jj, b):
                for u in range(CHK // 16):
                    src_v[jj, pl.ds(u * 16, 16)] = (
                        src_v[jj, pl.ds(u * 16, 16)] + off)
                pltpu.async_copy(tbl4.at[src_v.at[jj]], bufs[b], gsems[b])

            def scale(jj, b):
                rv = jnp.full((16,), jj, jnp.int32)

                def body(k, _):
                    ab = plsc.load_gather(
                        alpha_v, [rv, jnp.full((16,), k, jnp.int32)])
                    for u in range(H // 16):
                        bufs[b][k, pl.ds(u * 16, 16)] = (
                            bufs[b][k, pl.ds(u * 16, 16)] * ab)
                    return 0
                lax.fori_loop(0, CHK, body, 0, unroll=8)

            for b in range(_NB):
                issue_gather(b, b)

            def outer(g, _):
                for b in range(_NB):
                    jj = _NB * g + b
                    pltpu.make_async_copy(
                        tbl4.at[src_v.at[jj]], bufs[b], gsems[b]).wait()
                    scale(jj, b)
                    pltpu.async_copy(bufs[b], accum_sp.at[dst_v.at[jj]],
                                     ssems[b], add=True)

                @pl.when(g < _NH // _NB - 1)
                def _recycle():
                    for b in range(_NB):
                        jj = _NB * g + b
                        pltpu.make_async_copy(
                            bufs[b], accum_sp.at[dst_v.at[jj]],
                            ssems[b]).wait()
                        issue_gather(jj + _NB, b)
                return 0

            lax.fori_loop(0, _NH // _NB, outer, 0)
            for b in range(_NB):
                jj = _NH - _NB + b
                pltpu.make_async_copy(
                    bufs[b], accum_sp.at[dst_v.at[jj]], ssems[b]).wait()

    plsc.subcore_barrier()
    for i in range(5):
        sz = CHK if i < 4 else RPT - 4 * CHK
        pltpu.sync_copy(accum_sp.at[pl.ds(row0 + i * CHK, sz)],
                        outh.at[c, pl.ds(row0 + i * CHK, sz)])


_kb = functools.partial(
    pl.kernel,
    out_type=jax.ShapeDtypeStruct((NC, NP, H), jnp.float32),
    mesh=_MESH,
    compiler_params=_SC_PARAMS,
    scratch_types=(
        [pltpu.VMEM_SHARED((NP, H), jnp.float32),
         pltpu.VMEM((_NH, CHK), jnp.int32),
         pltpu.VMEM((_NH, CHK), jnp.int32),
         pltpu.VMEM((_NH, CHK), jnp.float32)]
        + [pltpu.VMEM((CHK, H), jnp.float32)] * _NB
        + [pltpu.SemaphoreType.DMA] * (2 * _NB)
    ),
)(_kb_body)


def kernel(h, edge_index0, edge_index1, W0, al0, ar0, W1, al1, ar1):
    Ws = jnp.stack([W0, W1])
    As = jnp.stack([jnp.stack([al0, ar0]), jnp.stack([al1, ar1])])
    tbl, sc = _tc_feat(h, Ws, As)

    pad = (jnp.arange(PADN, dtype=jnp.int32) * 97) % N

    def prep(ei):
        sfull = jnp.concatenate([ei[0], pad]).reshape(NS, NR, CHK)
        dfull = jnp.concatenate([ei[1], pad]).reshape(NS, NR, CHK)
        return sfull, dfull

    s0, d0 = prep(edge_index0)
    s1, d1 = prep(edge_index1)
    srcp = jnp.stack([s0, s1])
    dstp = jnp.stack([d0, d1])

    alpha = _ka(sc[:, 0], sc[:, 1], sc[:, 2], sc[:, 3], srcp, dstp)
    outh = _kb(tbl.reshape(4 * N, H), srcp, dstp, alpha)
    return jnp.concatenate([outh[0, :N], outh[1, :N]], axis=1)
